# baseline (speedup 1.0000x reference)
import functools

import jax
import jax.numpy as jnp
from jax import lax
from jax.experimental import pallas as pl
from jax.experimental.pallas import tpu as pltpu
from jax.experimental.pallas import tpu_sc as plsc

_N_TOKENS = 2048
_N_EMBD = 768
_N_EXP = 8
_TOP_K = 2
_CAP = 640
_HIDDEN = 4 * _N_EMBD
_NROWS = _N_EXP * _CAP
_TRASH = _NROWS
_BUF_ROWS = _NROWS + 8
_HB = 3072
_NPAIR = _TOP_K * _N_TOKENS
_NWORKERS = 32
_CHUNK = _NPAIR // _NWORKERS
_TCHUNK = _N_TOKENS // _NWORKERS


def _gelu(x):
    return 0.5 * x * (1.0 + jnp.tanh(jnp.sqrt(2.0 / jnp.pi) * (x + 0.044715 * x ** 3)))


def _router_body(x_ref, wg_ref, dst_ref, src_ref, w_ref):
    x = x_ref[...]
    wg = wg_ref[...]
    logits = jnp.dot(x, wg, preferred_element_type=jnp.float32)
    col = lax.broadcasted_iota(jnp.int32, logits.shape, 1)
    m0 = jnp.max(logits, axis=1, keepdims=True)
    e0 = jnp.min(jnp.where(logits == m0, col, _N_EXP), axis=1, keepdims=True)
    oh0 = col == e0
    l2 = jnp.where(oh0, -jnp.inf, logits)
    m1 = jnp.max(l2, axis=1, keepdims=True)
    e1 = jnp.min(jnp.where(l2 == m1, col, _N_EXP), axis=1, keepdims=True)
    oh1 = col == e1
    b = jnp.exp(m1 - m0)
    g0 = 1.0 / (1.0 + b)
    g1 = b / (1.0 + b)
    cnt = (oh0 | oh1).astype(jnp.float32)
    c = cnt
    s = 1
    while s < _N_TOKENS:
        c = c + jnp.concatenate(
            [jnp.zeros((s, _N_EXP), jnp.float32), c[:-s]], axis=0)
        s *= 2
    cex = c - cnt
    pos0 = jnp.sum(jnp.where(oh0, cex, 0.0), axis=1, keepdims=True).astype(jnp.int32)
    pos1 = jnp.sum(jnp.where(oh1, cex, 0.0), axis=1, keepdims=True).astype(jnp.int32)
    keep0 = pos0 < _CAP
    keep1 = pos1 < _CAP
    dst0 = jnp.where(keep0, e0 * _CAP + pos0, _TRASH)
    dst1 = jnp.where(keep1, e1 * _CAP + pos1, _TRASH)
    src0 = e0 * _CAP + jnp.minimum(pos0, _CAP - 1)
    src1 = e1 * _CAP + jnp.minimum(pos1, _CAP - 1)
    w0 = g0 * keep0.astype(jnp.float32)
    w1 = g1 * keep1.astype(jnp.float32)
    dst_ref[...] = jnp.concatenate([dst0, dst1], axis=0)
    src_ref[...] = jnp.concatenate([src0, src1], axis=0)
    w_ref[...] = jnp.concatenate([w0, w1], axis=1)


def _router(x, w_g):
    return pl.pallas_call(
        _router_body,
        out_shape=(
            jax.ShapeDtypeStruct((_NPAIR, 1), jnp.int32),
            jax.ShapeDtypeStruct((_NPAIR, 1), jnp.int32),
            jax.ShapeDtypeStruct((_N_TOKENS, _TOP_K), jnp.float32),
        ),
    )(x, w_g)


def _mlp_body(xin_ref, w1_ref, w2_ref, out_ref):
    xb = xin_ref[...].astype(jnp.bfloat16)
    w1 = w1_ref[0].astype(jnp.bfloat16)
    h = jnp.dot(xb, w1, preferred_element_type=jnp.float32)
    h = _gelu(h.astype(jnp.bfloat16))
    w2 = w2_ref[0].astype(jnp.bfloat16)
    out_ref[...] = jnp.dot(h, w2, preferred_element_type=jnp.float32)


def _mlp(buf, c_fc, c_proj):
    return pl.pallas_call(
        _mlp_body,
        grid=(_N_EXP,),
        in_specs=[
            pl.BlockSpec((_CAP, _N_EMBD), lambda e: (e, 0)),
            pl.BlockSpec((1, _N_EMBD, _HB), lambda e: (e, 0, 0)),
            pl.BlockSpec((1, _HB, _N_EMBD), lambda e: (e, 0, 0)),
        ],
        out_specs=pl.BlockSpec((_CAP, _N_EMBD), lambda e: (e, 0)),
        out_shape=jax.ShapeDtypeStruct((_NROWS, _N_EMBD), jnp.float32),
        compiler_params=pltpu.CompilerParams(
            dimension_semantics=("arbitrary",),
            vmem_limit_bytes=64 * 1024 * 1024),
    )(buf, c_fc, c_proj)


def _combine_body(ga_ref, gb_ref, w_ref, out_ref):
    w = w_ref[...]
    out_ref[...] = ga_ref[...] * w[:, 0:1] + gb_ref[...] * w[:, 1:2]


def _combine(g, ws):
    blk = 512
    nb = _N_TOKENS // blk
    return pl.pallas_call(
        _combine_body,
        grid=(nb,),
        in_specs=[
            pl.BlockSpec((blk, _N_EMBD), lambda i: (i, 0)),
            pl.BlockSpec((blk, _N_EMBD), lambda i: (i + nb, 0)),
            pl.BlockSpec((blk, _TOP_K), lambda i: (i, 0)),
        ],
        out_specs=pl.BlockSpec((blk, _N_EMBD), lambda i: (i, 0)),
        out_shape=jax.ShapeDtypeStruct((_N_TOKENS, _N_EMBD), jnp.float32),
    )(g, g, ws)


def _sc_dispatch(x, dst_flat):
    mesh = plsc.VectorSubcoreMesh(core_axis_name="c", subcore_axis_name="s")

    @functools.partial(
        pl.kernel,
        out_type=jax.ShapeDtypeStruct((_BUF_ROWS, _N_EMBD), jnp.float32),
        mesh=mesh,
        scratch_types=[
            pltpu.VMEM((_TCHUNK,), jnp.int32),
            pltpu.VMEM((_TCHUNK,), jnp.int32),
            pltpu.VMEM((_TCHUNK, _N_EMBD), jnp.float32),
            pltpu.SemaphoreType.DMA,
            pltpu.SemaphoreType.DMA,
        ],
    )
    def k(x_hbm, idx_hbm, buf_hbm, idx0_v, idx1_v, rows_v, sem0, sem1):
        wid = lax.axis_index("s") * 2 + lax.axis_index("c")
        base = wid * _TCHUNK
        pltpu.sync_copy(idx_hbm.at[pl.ds(base, _TCHUNK)], idx0_v)
        pltpu.sync_copy(idx_hbm.at[pl.ds(base + _N_TOKENS, _TCHUNK)], idx1_v)
        pltpu.sync_copy(x_hbm.at[pl.ds(base, _TCHUNK)], rows_v)
        cp0 = pltpu.make_async_copy(rows_v, buf_hbm.at[idx0_v], sem0)
        cp0.start()
        cp1 = pltpu.make_async_copy(rows_v, buf_hbm.at[idx1_v], sem1)
        cp1.start()
        cp0.wait()
        cp1.wait()

    return k(x, dst_flat)


def _sc_gather(eo, src_flat):
    mesh = plsc.VectorSubcoreMesh(core_axis_name="c", subcore_axis_name="s")

    @functools.partial(
        pl.kernel,
        out_type=jax.ShapeDtypeStruct((_NPAIR, _N_EMBD), jnp.float32),
        mesh=mesh,
        scratch_types=[
            pltpu.VMEM((_CHUNK,), jnp.int32),
            pltpu.VMEM((_CHUNK, _N_EMBD), jnp.float32),
            pltpu.SemaphoreType.DMA,
        ],
    )
    def k(eo_hbm, idx_hbm, out_hbm, idx_v, rows_v, sem):
        wid = lax.axis_index("s") * 2 + lax.axis_index("c")
        base = wid * _CHUNK
        pltpu.sync_copy(idx_hbm.at[pl.ds(base, _CHUNK)], idx_v)
        pltpu.async_copy(eo_hbm.at[idx_v], rows_v, sem).wait()
        pltpu.sync_copy(rows_v, out_hbm.at[pl.ds(base, _CHUNK)])

    return k(eo, src_flat)


def kernel(x, w_g, c_fc, c_proj):
    dstf, srcf, ws = _router(x, w_g)
    dst_flat = dstf.reshape(_NPAIR)
    src_flat = srcf.reshape(_NPAIR)
    buf = _sc_dispatch(x, dst_flat)
    eo = _mlp(buf, c_fc, c_proj)
    g = _sc_gather(eo, src_flat)
    return _combine(g, ws)


# speedup vs baseline: 1.0015x; 1.0015x over previous
import functools

import jax
import jax.numpy as jnp
from jax import lax
from jax.experimental import pallas as pl
from jax.experimental.pallas import tpu as pltpu
from jax.experimental.pallas import tpu_sc as plsc

_N_TOKENS = 2048
_N_EMBD = 768
_N_EXP = 8
_TOP_K = 2
_CAP = 640
_HIDDEN = 4 * _N_EMBD
_NROWS = _N_EXP * _CAP
_TRASH = _NROWS
_BUF_ROWS = _NROWS + 8
_HB = 3072
_NPAIR = _TOP_K * _N_TOKENS
_NWORKERS = 32
_CHUNK = _NPAIR // _NWORKERS
_TCHUNK = _N_TOKENS // _NWORKERS


def _gelu(x):
    return 0.5 * x * (1.0 + jnp.tanh(jnp.sqrt(2.0 / jnp.pi) * (x + 0.044715 * x ** 3)))


def _router_body(x_ref, wg_ref, dst_ref, src_ref, w_ref):
    x = x_ref[...]
    wg = wg_ref[...]
    logits = jnp.dot(x, wg, preferred_element_type=jnp.float32)
    col = lax.broadcasted_iota(jnp.int32, logits.shape, 1)
    m0 = jnp.max(logits, axis=1, keepdims=True)
    e0 = jnp.min(jnp.where(logits == m0, col, _N_EXP), axis=1, keepdims=True)
    oh0 = col == e0
    l2 = jnp.where(oh0, -jnp.inf, logits)
    m1 = jnp.max(l2, axis=1, keepdims=True)
    e1 = jnp.min(jnp.where(l2 == m1, col, _N_EXP), axis=1, keepdims=True)
    oh1 = col == e1
    b = jnp.exp(m1 - m0)
    g0 = 1.0 / (1.0 + b)
    g1 = b / (1.0 + b)
    cnt = (oh0 | oh1).astype(jnp.float32)
    c = cnt
    s = 1
    while s < _N_TOKENS:
        c = c + jnp.concatenate(
            [jnp.zeros((s, _N_EXP), jnp.float32), c[:-s]], axis=0)
        s *= 2
    cex = c - cnt
    pos0 = jnp.sum(jnp.where(oh0, cex, 0.0), axis=1, keepdims=True).astype(jnp.int32)
    pos1 = jnp.sum(jnp.where(oh1, cex, 0.0), axis=1, keepdims=True).astype(jnp.int32)
    keep0 = pos0 < _CAP
    keep1 = pos1 < _CAP
    dst0 = jnp.where(keep0, e0 * _CAP + pos0, _TRASH)
    dst1 = jnp.where(keep1, e1 * _CAP + pos1, _TRASH)
    src0 = e0 * _CAP + jnp.minimum(pos0, _CAP - 1)
    src1 = e1 * _CAP + jnp.minimum(pos1, _CAP - 1)
    w0 = g0 * keep0.astype(jnp.float32)
    w1 = g1 * keep1.astype(jnp.float32)
    dst_ref[...] = jnp.concatenate([dst0, dst1], axis=0)
    src_ref[...] = jnp.concatenate([src0, src1], axis=0)
    w_ref[...] = jnp.concatenate([w0, w1], axis=1)


def _router(x, w_g):
    return pl.pallas_call(
        _router_body,
        out_shape=(
            jax.ShapeDtypeStruct((_NPAIR, 1), jnp.int32),
            jax.ShapeDtypeStruct((_NPAIR, 1), jnp.int32),
            jax.ShapeDtypeStruct((_N_TOKENS, _TOP_K), jnp.float32),
        ),
    )(x, w_g)


def _mlp_body(xin_ref, w1_ref, w2_ref, out_ref):
    xb = xin_ref[...].astype(jnp.bfloat16)
    w1 = w1_ref[0].astype(jnp.bfloat16)
    h = jnp.dot(xb, w1, preferred_element_type=jnp.float32)
    h = _gelu(h.astype(jnp.bfloat16))
    w2 = w2_ref[0].astype(jnp.bfloat16)
    out_ref[...] = jnp.dot(h, w2, preferred_element_type=jnp.float32)


def _mlp(buf, c_fc, c_proj):
    return pl.pallas_call(
        _mlp_body,
        grid=(_N_EXP,),
        in_specs=[
            pl.BlockSpec((_CAP, _N_EMBD), lambda e: (e, 0)),
            pl.BlockSpec((1, _N_EMBD, _HB), lambda e: (e, 0, 0)),
            pl.BlockSpec((1, _HB, _N_EMBD), lambda e: (e, 0, 0)),
        ],
        out_specs=pl.BlockSpec((_CAP, _N_EMBD), lambda e: (e, 0)),
        out_shape=jax.ShapeDtypeStruct((_NROWS, _N_EMBD), jnp.float32),
        compiler_params=pltpu.CompilerParams(
            dimension_semantics=("arbitrary",)),
    )(buf, c_fc, c_proj)


def _combine_body(ga_ref, gb_ref, w_ref, out_ref):
    w = w_ref[...]
    out_ref[...] = ga_ref[...] * w[:, 0:1] + gb_ref[...] * w[:, 1:2]


def _combine(g, ws):
    blk = 512
    nb = _N_TOKENS // blk
    return pl.pallas_call(
        _combine_body,
        grid=(nb,),
        in_specs=[
            pl.BlockSpec((blk, _N_EMBD), lambda i: (i, 0)),
            pl.BlockSpec((blk, _N_EMBD), lambda i: (i + nb, 0)),
            pl.BlockSpec((blk, _TOP_K), lambda i: (i, 0)),
        ],
        out_specs=pl.BlockSpec((blk, _N_EMBD), lambda i: (i, 0)),
        out_shape=jax.ShapeDtypeStruct((_N_TOKENS, _N_EMBD), jnp.float32),
    )(g, g, ws)


def _sc_dispatch(x, dst_flat):
    mesh = plsc.VectorSubcoreMesh(core_axis_name="c", subcore_axis_name="s")

    @functools.partial(
        pl.kernel,
        out_type=jax.ShapeDtypeStruct((_BUF_ROWS, _N_EMBD), jnp.float32),
        mesh=mesh,
        scratch_types=[
            pltpu.VMEM((_TCHUNK,), jnp.int32),
            pltpu.VMEM((_TCHUNK,), jnp.int32),
            pltpu.VMEM((_TCHUNK, _N_EMBD), jnp.float32),
            pltpu.SemaphoreType.DMA,
            pltpu.SemaphoreType.DMA,
        ],
    )
    def k(x_hbm, idx_hbm, buf_hbm, idx0_v, idx1_v, rows_v, sem0, sem1):
        wid = lax.axis_index("s") * 2 + lax.axis_index("c")
        base = wid * _TCHUNK
        pltpu.sync_copy(idx_hbm.at[pl.ds(base, _TCHUNK)], idx0_v)
        pltpu.sync_copy(idx_hbm.at[pl.ds(base + _N_TOKENS, _TCHUNK)], idx1_v)
        pltpu.sync_copy(x_hbm.at[pl.ds(base, _TCHUNK)], rows_v)
        cp0 = pltpu.make_async_copy(rows_v, buf_hbm.at[idx0_v], sem0)
        cp0.start()
        cp1 = pltpu.make_async_copy(rows_v, buf_hbm.at[idx1_v], sem1)
        cp1.start()
        cp0.wait()
        cp1.wait()

    return k(x, dst_flat)


def _sc_gather(eo, src_flat):
    mesh = plsc.VectorSubcoreMesh(core_axis_name="c", subcore_axis_name="s")

    @functools.partial(
        pl.kernel,
        out_type=jax.ShapeDtypeStruct((_NPAIR, _N_EMBD), jnp.float32),
        mesh=mesh,
        scratch_types=[
            pltpu.VMEM((_CHUNK,), jnp.int32),
            pltpu.VMEM((_CHUNK, _N_EMBD), jnp.float32),
            pltpu.SemaphoreType.DMA,
        ],
    )
    def k(eo_hbm, idx_hbm, out_hbm, idx_v, rows_v, sem):
        wid = lax.axis_index("s") * 2 + lax.axis_index("c")
        base = wid * _CHUNK
        pltpu.sync_copy(idx_hbm.at[pl.ds(base, _CHUNK)], idx_v)
        pltpu.async_copy(eo_hbm.at[idx_v], rows_v, sem).wait()
        pltpu.sync_copy(rows_v, out_hbm.at[pl.ds(base, _CHUNK)])

    return k(eo, src_flat)


def kernel(x, w_g, c_fc, c_proj):
    dstf, srcf, ws = _router(x, w_g)
    dst_flat = dstf.reshape(_NPAIR)
    src_flat = srcf.reshape(_NPAIR)
    buf = _sc_dispatch(x, dst_flat)
    eo = _mlp(buf, c_fc, c_proj)
    g = _sc_gather(eo, src_flat)
    return _combine(g, ws)
